# baseline (speedup 1.0000x reference)
import functools

import jax
import jax.numpy as jnp
import numpy as np
from jax import lax
from jax.experimental import pallas as pl
from jax.experimental.pallas import tpu as pltpu
from jax.experimental.pallas import tpu_sc as plsc

B = 2
N = 16384
C = 256
NPTS = 512
NV = B * NPTS
MLP_HID = 128
SA_HID = 128
SA_OUT = 256
AGG_OUT = 512
NS0 = 16
NS1 = 32
R0SQ = 0.8 * 0.8
R1SQ = 1.6 * 1.6
NLANE = 16
SUPER = 16
NWORK = 32
VPW = NV // NWORK
BNSCALE = 1.0 / np.sqrt(1.0 + 1e-5)


def _vote_kernel(feat_ref, sxyz_ref, w1t_ref, b1_ref, bn1g_ref, bn1b_ref,
                 w2t_ref, b2_ref,
                 w1x0t_ref, b10_ref, w1x1t_ref, b11_ref,
                 off_ref, vp_ref, t0_ref, t1_ref):
    hp = (jnp.dot(feat_ref[...], w1t_ref[...],
                  preferred_element_type=jnp.float32)
          + b1_ref[...])
    h = jnp.maximum(bn1g_ref[...] * hp / jnp.sqrt(jnp.float32(1.0 + 1e-5))
                    + bn1b_ref[...], 0.0)
    off = jnp.dot(h, w2t_ref[...], preferred_element_type=jnp.float32) + b2_ref[...]
    col = lax.broadcasted_iota(jnp.int32, (1, 3), 1)
    bounds = jnp.where(col == 2, 2.0, 3.0).astype(jnp.float32)
    lim = jnp.clip(off, -bounds, bounds)
    vp = sxyz_ref[...] + lim
    off_ref[...] = off
    vp_ref[...] = vp
    t0_ref[...] = b10_ref[...] - jnp.dot(vp, w1x0t_ref[...],
                                         preferred_element_type=jnp.float32)
    t1_ref[...] = b11_ref[...] - jnp.dot(vp, w1x1t_ref[...],
                                         preferred_element_type=jnp.float32)


def _run_votes(seed_feat, seed_xyz, w1t, b1, bn1g, bn1b, w2t, b2,
               w1x0t, b10, w1x1t, b11):
    return pl.pallas_call(
        _vote_kernel,
        out_shape=(
            jax.ShapeDtypeStruct((NV, 3), jnp.float32),
            jax.ShapeDtypeStruct((NV, 3), jnp.float32),
            jax.ShapeDtypeStruct((NV, SA_HID), jnp.float32),
            jax.ShapeDtypeStruct((NV, SA_HID), jnp.float32),
        ),
    )(seed_feat, seed_xyz, w1t, b1, bn1g, bn1b, w2t, b2,
      w1x0t, b10, w1x1t, b11)


def _proj_kernel(xyz_ref, feat_ref, w1x0t_ref, w1f0t_ref, w1x1t_ref, w1f1t_ref,
                 q0_ref, q1_ref):
    xyz = xyz_ref[...]
    ft = feat_ref[...]
    q0_ref[...] = (jnp.dot(ft, w1f0t_ref[...], preferred_element_type=jnp.float32)
                   + jnp.dot(xyz, w1x0t_ref[...], preferred_element_type=jnp.float32))
    q1_ref[...] = (jnp.dot(ft, w1f1t_ref[...], preferred_element_type=jnp.float32)
                   + jnp.dot(xyz, w1x1t_ref[...], preferred_element_type=jnp.float32))


def _run_proj(xyz_all, feats, w1x0t, w1f0t, w1x1t, w1f1t):
    blk = 2048
    grid = (B * N) // blk
    return pl.pallas_call(
        _proj_kernel,
        grid=(grid,),
        in_specs=[
            pl.BlockSpec((blk, 3), lambda i: (i, 0)),
            pl.BlockSpec((blk, C), lambda i: (i, 0)),
            pl.BlockSpec((3, SA_HID), lambda i: (0, 0)),
            pl.BlockSpec((C, SA_HID), lambda i: (0, 0)),
            pl.BlockSpec((3, SA_HID), lambda i: (0, 0)),
            pl.BlockSpec((C, SA_HID), lambda i: (0, 0)),
        ],
        out_specs=(
            pl.BlockSpec((blk, SA_HID), lambda i: (i, 0)),
            pl.BlockSpec((blk, SA_HID), lambda i: (i, 0)),
        ),
        out_shape=(
            jax.ShapeDtypeStruct((B * N, SA_HID), jnp.float32),
            jax.ShapeDtypeStruct((B * N, SA_HID), jnp.float32),
        ),
    )(xyz_all, feats, w1x0t, w1f0t, w1x1t, w1f1t)


_PCH = 1024
_NCH = N // _PCH
_ECH = 256


def _ball_tc_kernel(vp_ref, xyzt_ref, triu_ref, idx0_ref, idx1_ref):
    f32 = jnp.float32
    vp = vp_ref[0]
    tri = triu_ref[...]
    col16 = lax.broadcasted_iota(jnp.int32, (1, NS0), 1)
    col32 = lax.broadcasted_iota(jnp.int32, (1, NS1), 1)
    piota = lax.broadcasted_iota(jnp.int32, (1, _PCH), 1).astype(f32)

    def chunk_work(ch, carry):
        prev0, prev1, acc0, acc1 = carry
        pts = xyzt_ref[0, :, pl.ds(ch * _PCH, _PCH)]
        dx = vp[:, 0:1] - pts[0:1, :]
        dy = vp[:, 1:2] - pts[1:2, :]
        dz = vp[:, 2:3] - pts[2:3, :]
        d2 = dx * dx + dy * dy + dz * dz
        pidx = piota + (ch * _PCH)
        wi0 = (d2 <= R0SQ).astype(f32)
        wi1 = (d2 <= R1SQ).astype(f32)
        cc0 = jnp.sum(wi0, axis=1, keepdims=True)
        cc1 = jnp.sum(wi1, axis=1, keepdims=True)
        np0 = prev0 + cc0
        np1 = prev1 + cc1

        def bounds(prev, npost, cc, cap):
            gain = (prev < cap) & (cc > 0.0)
            lo = jnp.min(jnp.where(gain, prev, jnp.float32(cap))) + 1.0
            hi = jnp.max(jnp.where(gain, jnp.minimum(npost, jnp.float32(cap)),
                                   0.0))
            return lo.astype(jnp.int32), hi.astype(jnp.int32)

        lo0, hi0 = bounds(prev0, np0, cc0, NS0)
        lo1, hi1 = bounds(prev1, np1, cc1, NS1)

        def do_slots(accs):
            acc0, acc1 = accs
            cs0 = jnp.dot(wi0, tri, preferred_element_type=f32) + prev0
            cs1 = jnp.dot(wi1, tri, preferred_element_type=f32) + prev1
            t0 = jnp.where(wi0 > 0.0, cs0, 0.0)
            t1 = jnp.where(wi1 > 0.0, cs1, 0.0)

            def slot0(s, a):
                contrib = jnp.sum(
                    jnp.where(t0 == s.astype(f32), pidx, 0.0),
                    axis=1, keepdims=True)
                return a + jnp.where(col16 == s - 1, contrib, 0.0)

            def slot1(s, a):
                contrib = jnp.sum(
                    jnp.where(t1 == s.astype(f32), pidx, 0.0),
                    axis=1, keepdims=True)
                return a + jnp.where(col32 == s - 1, contrib, 0.0)

            return (lax.fori_loop(lo0, hi0 + 1, slot0, acc0),
                    lax.fori_loop(lo1, hi1 + 1, slot1, acc1))

        acc0, acc1 = lax.cond((hi0 >= lo0) | (hi1 >= lo1), do_slots,
                              lambda a: a, (acc0, acc1))
        return (np0, np1, acc0, acc1)

    def chunk(ch, carry):
        prev0, prev1 = carry[0], carry[1]
        active = (jnp.min(prev0) < NS0) | (jnp.min(prev1) < NS1)
        return lax.cond(active, lambda c: chunk_work(ch, c), lambda c: c,
                        carry)

    z1 = jnp.zeros((NPTS, 1), f32)
    carry = (z1, z1, jnp.zeros((NPTS, NS0), f32),
             jnp.zeros((NPTS, NS1), f32))
    for ch in range(_NCH):
        carry = chunk(ch, carry)
    cnt0, cnt1, acc0, acc1 = carry

    first0 = acc0[:, 0:1]
    out0 = jnp.where(col16.astype(f32) < cnt0, acc0, first0)
    first1 = acc1[:, 0:1]
    out1 = jnp.where(col32.astype(f32) < cnt1, acc1, first1)
    idx0_ref[...] = out0.astype(jnp.int32)[None]
    idx1_ref[...] = out1.astype(jnp.int32)[None]


def _ball_query_tc(vp3, xyzt, triu):
    nb = vp3.shape[0]
    return pl.pallas_call(
        _ball_tc_kernel,
        grid=(nb,),
        in_specs=[
            pl.BlockSpec((1, NPTS, 3), lambda b: (b, 0, 0)),
            pl.BlockSpec((1, 3, N), lambda b: (b, 0, 0)),
            pl.BlockSpec((_PCH, _PCH), lambda b: (0, 0)),
        ],
        out_specs=(
            pl.BlockSpec((1, NPTS, NS0), lambda b: (b, 0, 0)),
            pl.BlockSpec((1, NPTS, NS1), lambda b: (b, 0, 0)),
        ),
        out_shape=(
            jax.ShapeDtypeStruct((nb, NPTS, NS0), jnp.int32),
            jax.ShapeDtypeStruct((nb, NPTS, NS1), jnp.int32),
        ),
    )(vp3, xyzt, triu)


_GCHUNK = 128


def _gather_body(q0_hbm, q1_hbm, idx0_hbm, idx1_hbm, g0_hbm, g1_hbm,
                 idxv, rows0, rows1, sem0, sem1):
    wid = lax.axis_index("s") * 2 + lax.axis_index("c")
    n0 = idx0_hbm.shape[0] // NWORK
    n1 = idx1_hbm.shape[0] // NWORK
    pltpu.sync_copy(idx0_hbm.at[pl.ds(wid * n0, n0)], idxv.at[pl.ds(0, n0)])
    pltpu.sync_copy(idx1_hbm.at[pl.ds(wid * n1, n1)], idxv.at[pl.ds(n0, n1)])

    nch = (n0 + n1) // _GCHUNK

    def chunk_src(k):
        off = k * _GCHUNK
        if off < n0:
            return q0_hbm, g0_hbm, wid * n0 + off
        return q1_hbm, g1_hbm, wid * n1 + (off - n0)

    rows = [rows0, rows1]
    sems = [sem0, sem1]

    def fire(k):
        q, _, _ = chunk_src(k)
        bb = k % 2
        return pltpu.async_copy(q.at[idxv.at[pl.ds(k * _GCHUNK, _GCHUNK)]],
                                rows[bb], sems[bb])

    cp = fire(0)
    for k in range(nch):
        nxt = fire(k + 1) if k + 1 < nch else None
        cp.wait()
        _, gout, goff = chunk_src(k)
        pltpu.sync_copy(rows[k % 2], gout.at[pl.ds(goff, _GCHUNK)])
        cp = nxt


def _gather_sc(q0, q1, idx0, idx1):
    mesh = plsc.VectorSubcoreMesh(core_axis_name="c", subcore_axis_name="s",
                                  num_cores=2, num_subcores=16)
    f = pl.kernel(
        _gather_body,
        out_type=(
            jax.ShapeDtypeStruct((idx0.shape[0], SA_HID), jnp.float32),
            jax.ShapeDtypeStruct((idx1.shape[0], SA_HID), jnp.float32),
        ),
        mesh=mesh,
        scratch_types=[
            pltpu.VMEM(((idx0.shape[0] + idx1.shape[0]) // NWORK,), jnp.int32),
            pltpu.VMEM((_GCHUNK, SA_HID), jnp.float32),
            pltpu.VMEM((_GCHUNK, SA_HID), jnp.float32),
            pltpu.SemaphoreType.DMA,
            pltpu.SemaphoreType.DMA,
        ],
    )
    return f(q0, q1, idx0, idx1)


def _head_kernel(g0_ref, g1_ref, t0_ref, t1_ref, w20t_ref, b20_ref,
                 w21t_ref, b21_ref, wot_ref, bo_ref, out_ref):
    vt = t0_ref.shape[0]

    def branch(g_ref, t_ref, w2t_ref, b2_ref, ns):
        g = g_ref[...].reshape(vt, ns, SA_HID)
        h = jnp.maximum(g + t_ref[...][:, None, :], 0.0)
        a = jnp.dot(h.reshape(vt * ns, SA_HID), w2t_ref[...],
                    preferred_element_type=jnp.float32)
        h2 = jnp.maximum(a + b2_ref[...], 0.0).reshape(vt, ns, SA_OUT)
        p = h2[:, 0, :]
        for k in range(1, ns):
            p = jnp.maximum(p, h2[:, k, :])
        return p

    p0 = branch(g0_ref, t0_ref, w20t_ref, b20_ref, NS0)
    p1 = branch(g1_ref, t1_ref, w21t_ref, b21_ref, NS1)
    pc = jnp.concatenate([p0, p1], axis=-1)
    out_ref[...] = jnp.maximum(
        jnp.dot(pc, wot_ref[...], preferred_element_type=jnp.float32)
        + bo_ref[...], 0.0)


def _run_head(g0, g1, t0, t1, w20t, b20, w21t, b21, wot, bo):
    vt = 128
    nv = t0.shape[0]
    grid = nv // vt
    return pl.pallas_call(
        _head_kernel,
        grid=(grid,),
        in_specs=[
            pl.BlockSpec((vt * NS0, SA_HID), lambda i: (i, 0)),
            pl.BlockSpec((vt * NS1, SA_HID), lambda i: (i, 0)),
            pl.BlockSpec((vt, SA_HID), lambda i: (i, 0)),
            pl.BlockSpec((vt, SA_HID), lambda i: (i, 0)),
            pl.BlockSpec((SA_HID, SA_OUT), lambda i: (0, 0)),
            pl.BlockSpec((1, SA_OUT), lambda i: (0, 0)),
            pl.BlockSpec((SA_HID, SA_OUT), lambda i: (0, 0)),
            pl.BlockSpec((1, SA_OUT), lambda i: (0, 0)),
            pl.BlockSpec((2 * SA_OUT, AGG_OUT), lambda i: (0, 0)),
            pl.BlockSpec((1, AGG_OUT), lambda i: (0, 0)),
        ],
        out_specs=pl.BlockSpec((vt, AGG_OUT), lambda i: (i, 0)),
        out_shape=jax.ShapeDtypeStruct((nv, AGG_OUT), jnp.float32),
    )(g0, g1, t0, t1, w20t, b20, w21t, b21, wot, bo)


def kernel(point_coords, point_features, batch_size, vote_w1, vote_b1,
           vote_bn1_g, vote_bn1_b, vote_w2, vote_b2, sa0_w1, sa0_bn1_g,
           sa0_bn1_b, sa0_w2, sa0_bn2_g, sa0_bn2_b, sa1_w1, sa1_bn1_g,
           sa1_bn1_b, sa1_w2, sa1_bn2_g, sa1_bn2_b, out_w, out_bn_g,
           out_bn_b):
    f32 = jnp.float32
    s = jnp.asarray(BNSCALE, f32)

    w1v = vote_w1 * (vote_bn1_g * s)[:, None]
    b1v = (vote_bn1_g * s * vote_b1 + vote_bn1_b)[None, :]
    w1e0 = sa0_w1 * (sa0_bn1_g * s)[:, None]
    w1e1 = sa1_w1 * (sa1_bn1_g * s)[:, None]
    w2e0t = (sa0_w2 * (sa0_bn2_g * s)[:, None]).T
    w2e1t = (sa1_w2 * (sa1_bn2_g * s)[:, None]).T
    b20 = sa0_bn2_b[None, :]
    b21 = sa1_bn2_b[None, :]
    wot = (out_w * (out_bn_g * s)[:, None]).T
    bo = out_bn_b[None, :]

    xyz_all = point_coords[:, 1:4]
    feats = point_features
    seed_xyz = xyz_all.reshape(B, N, 3)[:, :NPTS].reshape(NV, 3)
    seed_feat = feats.reshape(B, N, C)[:, :NPTS].reshape(NV, C)

    off, vp, t0, t1 = _run_votes(
        seed_feat, seed_xyz, vote_w1.T, vote_b1[None, :],
        vote_bn1_g[None, :], vote_bn1_b[None, :], vote_w2.T,
        vote_b2[None, :],
        w1e0[:, :3].T, sa0_bn1_b[None, :], w1e1[:, :3].T, sa1_bn1_b[None, :])

    xyzt = jnp.transpose(xyz_all.reshape(B, N, 3), (0, 2, 1))
    triu = jnp.triu(jnp.ones((_PCH, _PCH), jnp.float32))
    q0, q1 = _run_proj(xyz_all, feats, w1e0[:, :3].T, w1e0[:, 3:].T,
                       w1e1[:, :3].T, w1e1[:, 3:].T)
    vp3 = vp.reshape(B, NPTS, 3)
    t03 = t0.reshape(B, NPTS, SA_HID)
    t13 = t1.reshape(B, NPTS, SA_HID)
    cfs = []
    for b in range(B):
        i0, i1 = _ball_query_tc(vp3[b:b + 1], xyzt[b:b + 1], triu)
        i0 = (i0 + b * N).reshape(-1)
        i1 = (i1 + b * N).reshape(-1)
        g0, g1 = _gather_sc(q0, q1, i0, i1)
        cfs.append(_run_head(g0, g1, t03[b], t13[b], w2e0t, b20, w2e1t,
                             b21, wot, bo))
    cf = jnp.concatenate(cfs, axis=0)

    bs_res = (jnp.asarray(batch_size) - B).astype(f32)
    ctr_idx = jnp.repeat(jnp.arange(B, dtype=f32), NPTS)[:, None] + bs_res
    ctr_offsets = jnp.concatenate([ctr_idx, off], axis=1)
    centers = jnp.concatenate([ctr_idx, vp], axis=1)
    centers_origin = jnp.concatenate([ctr_idx, seed_xyz], axis=1)
    return (ctr_offsets, centers, centers_origin, cf)


# speedup vs baseline: 1.1449x; 1.1449x over previous
import functools

import jax
import jax.numpy as jnp
import numpy as np
from jax import lax
from jax.experimental import pallas as pl
from jax.experimental.pallas import tpu as pltpu
from jax.experimental.pallas import tpu_sc as plsc

B = 2
N = 16384
C = 256
NPTS = 512
NV = B * NPTS
MLP_HID = 128
SA_HID = 128
SA_OUT = 256
AGG_OUT = 512
NS0 = 16
NS1 = 32
R0SQ = 0.8 * 0.8
R1SQ = 1.6 * 1.6
NLANE = 16
SUPER = 16
NWORK = 32
VPW = NV // NWORK
BNSCALE = 1.0 / np.sqrt(1.0 + 1e-5)


def _vote_kernel(feat_ref, sxyz_ref, w1t_ref, b1_ref, bn1g_ref, bn1b_ref,
                 w2t_ref, b2_ref,
                 w1x0t_ref, b10_ref, w1x1t_ref, b11_ref,
                 off_ref, vp_ref, t0_ref, t1_ref):
    hp = (jnp.dot(feat_ref[...], w1t_ref[...],
                  preferred_element_type=jnp.float32)
          + b1_ref[...])
    h = jnp.maximum(bn1g_ref[...] * hp / jnp.sqrt(jnp.float32(1.0 + 1e-5))
                    + bn1b_ref[...], 0.0)
    off = jnp.dot(h, w2t_ref[...], preferred_element_type=jnp.float32) + b2_ref[...]
    col = lax.broadcasted_iota(jnp.int32, (1, 3), 1)
    bounds = jnp.where(col == 2, 2.0, 3.0).astype(jnp.float32)
    lim = jnp.clip(off, -bounds, bounds)
    vp = sxyz_ref[...] + lim
    off_ref[...] = off
    vp_ref[...] = vp
    t0_ref[...] = b10_ref[...] - jnp.dot(vp, w1x0t_ref[...],
                                         preferred_element_type=jnp.float32)
    t1_ref[...] = b11_ref[...] - jnp.dot(vp, w1x1t_ref[...],
                                         preferred_element_type=jnp.float32)


def _run_votes(seed_feat, seed_xyz, w1t, b1, bn1g, bn1b, w2t, b2,
               w1x0t, b10, w1x1t, b11):
    return pl.pallas_call(
        _vote_kernel,
        out_shape=(
            jax.ShapeDtypeStruct((NV, 3), jnp.float32),
            jax.ShapeDtypeStruct((NV, 3), jnp.float32),
            jax.ShapeDtypeStruct((NV, SA_HID), jnp.float32),
            jax.ShapeDtypeStruct((NV, SA_HID), jnp.float32),
        ),
    )(seed_feat, seed_xyz, w1t, b1, bn1g, bn1b, w2t, b2,
      w1x0t, b10, w1x1t, b11)


def _proj_kernel(xyz_ref, feat_ref, w1x0t_ref, w1f0t_ref, w1x1t_ref, w1f1t_ref,
                 q0_ref, q1_ref):
    xyz = xyz_ref[...]
    ft = feat_ref[...]
    q0_ref[...] = (jnp.dot(ft, w1f0t_ref[...], preferred_element_type=jnp.float32)
                   + jnp.dot(xyz, w1x0t_ref[...], preferred_element_type=jnp.float32))
    q1_ref[...] = (jnp.dot(ft, w1f1t_ref[...], preferred_element_type=jnp.float32)
                   + jnp.dot(xyz, w1x1t_ref[...], preferred_element_type=jnp.float32))


def _run_proj(xyz_all, feats, w1x0t, w1f0t, w1x1t, w1f1t):
    blk = 2048
    grid = (B * N) // blk
    return pl.pallas_call(
        _proj_kernel,
        grid=(grid,),
        in_specs=[
            pl.BlockSpec((blk, 3), lambda i: (i, 0)),
            pl.BlockSpec((blk, C), lambda i: (i, 0)),
            pl.BlockSpec((3, SA_HID), lambda i: (0, 0)),
            pl.BlockSpec((C, SA_HID), lambda i: (0, 0)),
            pl.BlockSpec((3, SA_HID), lambda i: (0, 0)),
            pl.BlockSpec((C, SA_HID), lambda i: (0, 0)),
        ],
        out_specs=(
            pl.BlockSpec((blk, SA_HID), lambda i: (i, 0)),
            pl.BlockSpec((blk, SA_HID), lambda i: (i, 0)),
        ),
        out_shape=(
            jax.ShapeDtypeStruct((B * N, SA_HID), jnp.float32),
            jax.ShapeDtypeStruct((B * N, SA_HID), jnp.float32),
        ),
    )(xyz_all, feats, w1x0t, w1f0t, w1x1t, w1f1t)


_PCH = 1024
_NCH = N // _PCH
_ECH = 256


def _ball_tc_kernel(vp_ref, xyzt_ref, triu_ref, idx0_ref, idx1_ref):
    f32 = jnp.float32
    vp = vp_ref[0]
    tri = triu_ref[...]
    col16 = lax.broadcasted_iota(jnp.int32, (1, NS0), 1)
    col32 = lax.broadcasted_iota(jnp.int32, (1, NS1), 1)
    piota = lax.broadcasted_iota(jnp.int32, (1, _PCH), 1).astype(f32)

    def chunk_work(ch, carry):
        prev0, prev1, acc0, acc1 = carry
        pts = xyzt_ref[0, :, pl.ds(ch * _PCH, _PCH)]
        dx = vp[:, 0:1] - pts[0:1, :]
        dy = vp[:, 1:2] - pts[1:2, :]
        dz = vp[:, 2:3] - pts[2:3, :]
        d2 = dx * dx + dy * dy + dz * dz
        pidx = piota + (ch * _PCH)
        wi0 = (d2 <= R0SQ).astype(f32)
        wi1 = (d2 <= R1SQ).astype(f32)
        cc0 = jnp.sum(wi0, axis=1, keepdims=True)
        cc1 = jnp.sum(wi1, axis=1, keepdims=True)
        np0 = prev0 + cc0
        np1 = prev1 + cc1

        def bounds(prev, npost, cc, cap):
            gain = (prev < cap) & (cc > 0.0)
            lo = jnp.min(jnp.where(gain, prev, jnp.float32(cap))) + 1.0
            hi = jnp.max(jnp.where(gain, jnp.minimum(npost, jnp.float32(cap)),
                                   0.0))
            return lo.astype(jnp.int32), hi.astype(jnp.int32)

        lo0, hi0 = bounds(prev0, np0, cc0, NS0)
        lo1, hi1 = bounds(prev1, np1, cc1, NS1)

        def extract(wi, prev, lo, hi, col):
            def work(a):
                cs = jnp.dot(wi, tri, preferred_element_type=f32) + prev
                t = jnp.where(wi > 0.0, cs, 0.0)

                def slot(s, acc):
                    contrib = jnp.sum(
                        jnp.where(t == s.astype(f32), pidx, 0.0),
                        axis=1, keepdims=True)
                    return acc + jnp.where(col == s - 1, contrib, 0.0)

                return lax.fori_loop(lo, hi + 1, slot, a)

            return work

        acc0 = lax.cond(hi0 >= lo0, extract(wi0, prev0, lo0, hi0, col16),
                        lambda a: a, acc0)
        acc1 = lax.cond(hi1 >= lo1, extract(wi1, prev1, lo1, hi1, col32),
                        lambda a: a, acc1)
        return (np0, np1, acc0, acc1)

    def chunk(ch, carry):
        prev0, prev1 = carry[0], carry[1]
        active = (jnp.min(prev0) < NS0) | (jnp.min(prev1) < NS1)
        return lax.cond(active, lambda c: chunk_work(ch, c), lambda c: c,
                        carry)

    z1 = jnp.zeros((NPTS, 1), f32)
    cnt0, cnt1, acc0, acc1 = lax.fori_loop(
        0, _NCH, chunk,
        (z1, z1, jnp.zeros((NPTS, NS0), f32), jnp.zeros((NPTS, NS1), f32)))

    first0 = acc0[:, 0:1]
    out0 = jnp.where(col16.astype(f32) < cnt0, acc0, first0)
    first1 = acc1[:, 0:1]
    out1 = jnp.where(col32.astype(f32) < cnt1, acc1, first1)
    idx0_ref[...] = out0.astype(jnp.int32)[None]
    idx1_ref[...] = out1.astype(jnp.int32)[None]


def _ball_query_tc(vp3, xyzt, triu):
    nb = vp3.shape[0]
    return pl.pallas_call(
        _ball_tc_kernel,
        grid=(nb,),
        in_specs=[
            pl.BlockSpec((1, NPTS, 3), lambda b: (b, 0, 0)),
            pl.BlockSpec((1, 3, N), lambda b: (b, 0, 0)),
            pl.BlockSpec((_PCH, _PCH), lambda b: (0, 0)),
        ],
        out_specs=(
            pl.BlockSpec((1, NPTS, NS0), lambda b: (b, 0, 0)),
            pl.BlockSpec((1, NPTS, NS1), lambda b: (b, 0, 0)),
        ),
        out_shape=(
            jax.ShapeDtypeStruct((nb, NPTS, NS0), jnp.int32),
            jax.ShapeDtypeStruct((nb, NPTS, NS1), jnp.int32),
        ),
    )(vp3, xyzt, triu)


_GCHUNK = 128


def _gather_body(q0_hbm, q1_hbm, idx0_hbm, idx1_hbm, g0_hbm, g1_hbm,
                 idxv, rows0, rows1, sem0, sem1):
    wid = lax.axis_index("s") * 2 + lax.axis_index("c")
    n0 = idx0_hbm.shape[0] // NWORK
    n1 = idx1_hbm.shape[0] // NWORK
    pltpu.sync_copy(idx0_hbm.at[pl.ds(wid * n0, n0)], idxv.at[pl.ds(0, n0)])
    pltpu.sync_copy(idx1_hbm.at[pl.ds(wid * n1, n1)], idxv.at[pl.ds(n0, n1)])

    nch = (n0 + n1) // _GCHUNK

    def chunk_src(k):
        off = k * _GCHUNK
        if off < n0:
            return q0_hbm, g0_hbm, wid * n0 + off
        return q1_hbm, g1_hbm, wid * n1 + (off - n0)

    rows = [rows0, rows1]
    sems = [sem0, sem1]

    def fire(k):
        q, _, _ = chunk_src(k)
        bb = k % 2
        return pltpu.async_copy(q.at[idxv.at[pl.ds(k * _GCHUNK, _GCHUNK)]],
                                rows[bb], sems[bb])

    cp = fire(0)
    for k in range(nch):
        nxt = fire(k + 1) if k + 1 < nch else None
        cp.wait()
        _, gout, goff = chunk_src(k)
        pltpu.sync_copy(rows[k % 2], gout.at[pl.ds(goff, _GCHUNK)])
        cp = nxt


def _gather_sc(q0, q1, idx0, idx1):
    mesh = plsc.VectorSubcoreMesh(core_axis_name="c", subcore_axis_name="s",
                                  num_cores=2, num_subcores=16)
    f = pl.kernel(
        _gather_body,
        out_type=(
            jax.ShapeDtypeStruct((idx0.shape[0], SA_HID), jnp.float32),
            jax.ShapeDtypeStruct((idx1.shape[0], SA_HID), jnp.float32),
        ),
        mesh=mesh,
        scratch_types=[
            pltpu.VMEM(((idx0.shape[0] + idx1.shape[0]) // NWORK,), jnp.int32),
            pltpu.VMEM((_GCHUNK, SA_HID), jnp.float32),
            pltpu.VMEM((_GCHUNK, SA_HID), jnp.float32),
            pltpu.SemaphoreType.DMA,
            pltpu.SemaphoreType.DMA,
        ],
    )
    return f(q0, q1, idx0, idx1)


def _head_kernel(g0_ref, g1_ref, t0_ref, t1_ref, w20t_ref, b20_ref,
                 w21t_ref, b21_ref, wot_ref, bo_ref, out_ref):
    vt = t0_ref.shape[0]

    def branch(g_ref, t_ref, w2t_ref, b2_ref, ns):
        g = g_ref[...].reshape(vt, ns, SA_HID)
        h = jnp.maximum(g + t_ref[...][:, None, :], 0.0)
        a = jnp.dot(h.reshape(vt * ns, SA_HID), w2t_ref[...],
                    preferred_element_type=jnp.float32)
        h2 = jnp.maximum(a + b2_ref[...], 0.0).reshape(vt, ns, SA_OUT)
        p = h2[:, 0, :]
        for k in range(1, ns):
            p = jnp.maximum(p, h2[:, k, :])
        return p

    p0 = branch(g0_ref, t0_ref, w20t_ref, b20_ref, NS0)
    p1 = branch(g1_ref, t1_ref, w21t_ref, b21_ref, NS1)
    pc = jnp.concatenate([p0, p1], axis=-1)
    out_ref[...] = jnp.maximum(
        jnp.dot(pc, wot_ref[...], preferred_element_type=jnp.float32)
        + bo_ref[...], 0.0)


def _run_head(g0, g1, t0, t1, w20t, b20, w21t, b21, wot, bo):
    vt = 128
    nv = t0.shape[0]
    grid = nv // vt
    return pl.pallas_call(
        _head_kernel,
        grid=(grid,),
        in_specs=[
            pl.BlockSpec((vt * NS0, SA_HID), lambda i: (i, 0)),
            pl.BlockSpec((vt * NS1, SA_HID), lambda i: (i, 0)),
            pl.BlockSpec((vt, SA_HID), lambda i: (i, 0)),
            pl.BlockSpec((vt, SA_HID), lambda i: (i, 0)),
            pl.BlockSpec((SA_HID, SA_OUT), lambda i: (0, 0)),
            pl.BlockSpec((1, SA_OUT), lambda i: (0, 0)),
            pl.BlockSpec((SA_HID, SA_OUT), lambda i: (0, 0)),
            pl.BlockSpec((1, SA_OUT), lambda i: (0, 0)),
            pl.BlockSpec((2 * SA_OUT, AGG_OUT), lambda i: (0, 0)),
            pl.BlockSpec((1, AGG_OUT), lambda i: (0, 0)),
        ],
        out_specs=pl.BlockSpec((vt, AGG_OUT), lambda i: (i, 0)),
        out_shape=jax.ShapeDtypeStruct((nv, AGG_OUT), jnp.float32),
    )(g0, g1, t0, t1, w20t, b20, w21t, b21, wot, bo)


def kernel(point_coords, point_features, batch_size, vote_w1, vote_b1,
           vote_bn1_g, vote_bn1_b, vote_w2, vote_b2, sa0_w1, sa0_bn1_g,
           sa0_bn1_b, sa0_w2, sa0_bn2_g, sa0_bn2_b, sa1_w1, sa1_bn1_g,
           sa1_bn1_b, sa1_w2, sa1_bn2_g, sa1_bn2_b, out_w, out_bn_g,
           out_bn_b):
    f32 = jnp.float32
    s = jnp.asarray(BNSCALE, f32)

    w1v = vote_w1 * (vote_bn1_g * s)[:, None]
    b1v = (vote_bn1_g * s * vote_b1 + vote_bn1_b)[None, :]
    w1e0 = sa0_w1 * (sa0_bn1_g * s)[:, None]
    w1e1 = sa1_w1 * (sa1_bn1_g * s)[:, None]
    w2e0t = (sa0_w2 * (sa0_bn2_g * s)[:, None]).T
    w2e1t = (sa1_w2 * (sa1_bn2_g * s)[:, None]).T
    b20 = sa0_bn2_b[None, :]
    b21 = sa1_bn2_b[None, :]
    wot = (out_w * (out_bn_g * s)[:, None]).T
    bo = out_bn_b[None, :]

    xyz_all = point_coords[:, 1:4]
    feats = point_features
    seed_xyz = xyz_all.reshape(B, N, 3)[:, :NPTS].reshape(NV, 3)
    seed_feat = feats.reshape(B, N, C)[:, :NPTS].reshape(NV, C)

    off, vp, t0, t1 = _run_votes(
        seed_feat, seed_xyz, vote_w1.T, vote_b1[None, :],
        vote_bn1_g[None, :], vote_bn1_b[None, :], vote_w2.T,
        vote_b2[None, :],
        w1e0[:, :3].T, sa0_bn1_b[None, :], w1e1[:, :3].T, sa1_bn1_b[None, :])

    xyzt = jnp.transpose(xyz_all.reshape(B, N, 3), (0, 2, 1))
    triu = jnp.triu(jnp.ones((_PCH, _PCH), jnp.float32))
    q0, q1 = _run_proj(xyz_all, feats, w1e0[:, :3].T, w1e0[:, 3:].T,
                       w1e1[:, :3].T, w1e1[:, 3:].T)
    vp3 = vp.reshape(B, NPTS, 3)
    t03 = t0.reshape(B, NPTS, SA_HID)
    t13 = t1.reshape(B, NPTS, SA_HID)
    cfs = []
    for b in range(B):
        i0, i1 = _ball_query_tc(vp3[b:b + 1], xyzt[b:b + 1], triu)
        i0 = (i0 + b * N).reshape(-1)
        i1 = (i1 + b * N).reshape(-1)
        g0, g1 = _gather_sc(q0, q1, i0, i1)
        cfs.append(_run_head(g0, g1, t03[b], t13[b], w2e0t, b20, w2e1t,
                             b21, wot, bo))
    cf = jnp.concatenate(cfs, axis=0)

    bs_res = (jnp.asarray(batch_size) - B).astype(f32)
    ctr_idx = jnp.repeat(jnp.arange(B, dtype=f32), NPTS)[:, None] + bs_res
    ctr_offsets = jnp.concatenate([ctr_idx, off], axis=1)
    centers = jnp.concatenate([ctr_idx, vp], axis=1)
    centers_origin = jnp.concatenate([ctr_idx, seed_xyz], axis=1)
    return (ctr_offsets, centers, centers_origin, cf)
